# baseline (speedup 1.0000x reference)
import math

import jax
import jax.numpy as jnp
from jax.experimental import pallas as pl

_N_NODES = 10000
_NODE_DIM = 128
_EDGE_DIM = 16
_HID = 256
_HEADS = 8
_OUTC = _HID // _HEADS


def _ln_block(h, g, b):
    mu = jnp.mean(h, axis=-1, keepdims=True)
    var = jnp.mean((h - mu) * (h - mu), axis=-1, keepdims=True)
    return (h - mu) * jax.lax.rsqrt(var + 1e-5) * g + b


def _mm_kern(x_ref, w_ref, b_ref, o_ref):
    o_ref[...] = (
        jnp.dot(x_ref[...], w_ref[...], preferred_element_type=jnp.float32, precision=jax.lax.Precision.HIGHEST)
        + b_ref[...]
    )


def _mm_bias(x, W, b, bm=512):
    M, K = x.shape
    N = W.shape[1]
    return pl.pallas_call(
        _mm_kern,
        grid=(pl.cdiv(M, bm),),
        in_specs=[
            pl.BlockSpec((bm, K), lambda i: (i, 0)),
            pl.BlockSpec((K, N), lambda i: (0, 0)),
            pl.BlockSpec((1, N), lambda i: (0, 0)),
        ],
        out_specs=pl.BlockSpec((bm, N), lambda i: (i, 0)),
        out_shape=jax.ShapeDtypeStruct((M, N), jnp.float32),
    )(x, W, b.reshape(1, N))


def _gelu_exact(v):
    return 0.5 * v * (1.0 + jax.lax.erf(v * 0.7071067811865476))


def _layer_kern(node_ref, attn_ref, n1g, n1b, w1, b1, w2, b2, n2g, n2b, o_ref):
    h = _ln_block(node_ref[...] + attn_ref[...], n1g[...], n1b[...])
    f = _gelu_exact(
        jnp.dot(h, w1[...], preferred_element_type=jnp.float32, precision=jax.lax.Precision.HIGHEST) + b1[...]
    )
    f = jnp.dot(f, w2[...], preferred_element_type=jnp.float32, precision=jax.lax.Precision.HIGHEST) + b2[...]
    o_ref[...] = _ln_block(h + f, n2g[...], n2b[...])


def _layer_dense(node, attn, lp, bm=256):
    M = node.shape[0]
    H = _HID
    row = lambda v: v.reshape(1, -1)
    full = lambda shape: pl.BlockSpec(shape, lambda i: (0, 0))
    return pl.pallas_call(
        _layer_kern,
        grid=(pl.cdiv(M, bm),),
        in_specs=[
            pl.BlockSpec((bm, H), lambda i: (i, 0)),
            pl.BlockSpec((bm, H), lambda i: (i, 0)),
            full((1, H)),
            full((1, H)),
            full((H, 4 * H)),
            full((1, 4 * H)),
            full((4 * H, H)),
            full((1, H)),
            full((1, H)),
            full((1, H)),
        ],
        out_specs=pl.BlockSpec((bm, H), lambda i: (i, 0)),
        out_shape=jax.ShapeDtypeStruct((M, H), jnp.float32),
    )(
        node,
        attn,
        row(lp["n1g"]),
        row(lp["n1b"]),
        lp["fW1"],
        row(lp["fb1"]),
        lp["fW2"],
        row(lp["fb2"]),
        row(lp["n2g"]),
        row(lp["n2b"]),
    )


def _lnmm_kern(x_ref, g_ref, b_ref, w_ref, bw_ref, o_ref):
    h = _ln_block(x_ref[...], g_ref[...], b_ref[...])
    o_ref[...] = (
        jnp.dot(h, w_ref[...], preferred_element_type=jnp.float32, precision=jax.lax.Precision.HIGHEST) + bw_ref[...]
    )


def _ln_mm(x, g, b, W, bw, bm=512):
    M, K = x.shape
    N = W.shape[1]
    full = lambda shape: pl.BlockSpec(shape, lambda i: (0, 0))
    return pl.pallas_call(
        _lnmm_kern,
        grid=(pl.cdiv(M, bm),),
        in_specs=[
            pl.BlockSpec((bm, K), lambda i: (i, 0)),
            full((1, K)),
            full((1, K)),
            full((K, N)),
            full((1, N)),
        ],
        out_specs=pl.BlockSpec((bm, N), lambda i: (i, 0)),
        out_shape=jax.ShapeDtypeStruct((M, N), jnp.float32),
    )(x, g.reshape(1, K), b.reshape(1, K), W, bw.reshape(1, N))


def _time_embed(t):
    half = _HID // 2
    tf = jnp.clip(t, 0, 999).astype(jnp.float32)
    freqs = jnp.exp(
        jnp.arange(half, dtype=jnp.float32) * -(math.log(10000.0) / (half - 1))
    )
    ang = tf[:, None] * freqs[None, :]
    return jnp.concatenate([jnp.sin(ang), jnp.cos(ang)], axis=-1)


def _gatv2(xn, src, dst, e_emb, p):
    xl = _mm_bias(xn, p["Wl"], p["bl"]).reshape(-1, _HEADS, _OUTC)
    xr = _mm_bias(xn, p["Wr"], p["br"]).reshape(-1, _HEADS, _OUTC)
    e = _mm_bias(e_emb, p["We"], jnp.zeros((_HID,), jnp.float32)).reshape(
        -1, _HEADS, _OUTC
    )
    mask = src != dst
    mf = mask.astype(jnp.float32)
    cnt = jax.ops.segment_sum(mf, dst, num_segments=_N_NODES)
    loop_e = jax.ops.segment_sum(e * mf[:, None, None], dst, num_segments=_N_NODES)
    loop_e = jnp.where(
        cnt[:, None, None] > 0,
        loop_e / jnp.maximum(cnt, 1.0)[:, None, None],
        0.0,
    )
    m = jax.nn.leaky_relu(xl[src] + xr[dst] + e, 0.2)
    m_loop = jax.nn.leaky_relu(xl + xr + loop_e, 0.2)
    a = jnp.sum(m * p["att"][None], axis=-1)
    a_loop = jnp.sum(m_loop * p["att"][None], axis=-1)
    a_for_max = jnp.where(mask[:, None], a, -jnp.inf)
    amax = jnp.maximum(
        jax.ops.segment_max(a_for_max, dst, num_segments=_N_NODES), a_loop
    )
    a_safe = jnp.where(mask[:, None], a, amax[dst])
    ex = jnp.exp(a_safe - amax[dst]) * mf[:, None]
    ex_loop = jnp.exp(a_loop - amax)
    den = jax.ops.segment_sum(ex, dst, num_segments=_N_NODES) + ex_loop
    out = jax.ops.segment_sum(
        xl[src] * (ex / (den[dst] + 1e-16))[..., None], dst, num_segments=_N_NODES
    )
    out = out + xl * (ex_loop / (den + 1e-16))[..., None]
    return out.reshape(-1, _HID) + p["bias"]


def kernel(x, edge_index, edge_attr, batch, timesteps, params):
    xi = jnp.clip(x, 0, _NODE_DIM - 1)
    ei = jnp.clip(edge_attr, 0, _EDGE_DIM - 1)
    node = params["node_embed"][xi]
    eemb = params["edge_embed"][ei]
    te = _time_embed(timesteps)
    te = (
        jax.nn.gelu(te @ params["tm_W1"] + params["tm_b1"], approximate=False)
        @ params["tm_W2"]
        + params["tm_b2"]
    )
    node = node + te[batch]
    src = edge_index[0]
    dst = edge_index[1]
    for lp in params["layers"]:
        attn = _gatv2(node, src, dst, eemb, lp)
        node = _layer_dense(node, attn, lp)
    pred_node = _ln_mm(node, params["hn_g"], params["hn_b"], params["hn_W"], params["hn_bW"])
    ef = (node[src] + node[dst]) * 0.5
    pred_edge = _ln_mm(ef, params["he_g"], params["he_b"], params["he_W"], params["he_bW"])
    return (pred_node, pred_edge)


# speedup vs baseline: 1.0185x; 1.0185x over previous
import math

import jax
import jax.numpy as jnp
from jax.experimental import pallas as pl

_N_NODES = 10000
_NODE_DIM = 128
_EDGE_DIM = 16
_HID = 256
_HEADS = 8
_OUTC = _HID // _HEADS


def _ln_block(h, g, b):
    mu = jnp.mean(h, axis=-1, keepdims=True)
    var = jnp.mean((h - mu) * (h - mu), axis=-1, keepdims=True)
    return (h - mu) * jax.lax.rsqrt(var + 1e-5) * g + b


def _mm_kern(x_ref, w_ref, b_ref, o_ref):
    o_ref[...] = (
        jnp.dot(x_ref[...], w_ref[...], preferred_element_type=jnp.float32, precision=jax.lax.Precision.HIGHEST)
        + b_ref[...]
    )


def _mm_bias(x, W, b, bm=512):
    M, K = x.shape
    N = W.shape[1]
    return pl.pallas_call(
        _mm_kern,
        grid=(pl.cdiv(M, bm),),
        in_specs=[
            pl.BlockSpec((bm, K), lambda i: (i, 0)),
            pl.BlockSpec((K, N), lambda i: (0, 0)),
            pl.BlockSpec((1, N), lambda i: (0, 0)),
        ],
        out_specs=pl.BlockSpec((bm, N), lambda i: (i, 0)),
        out_shape=jax.ShapeDtypeStruct((M, N), jnp.float32),
    )(x, W, b.reshape(1, N))


def _gelu_exact(v):
    return 0.5 * v * (1.0 + jax.lax.erf(v * 0.7071067811865476))


def _layer_kern(node_ref, attn_ref, n1g, n1b, w1, b1, w2, b2, n2g, n2b, o_ref):
    h = _ln_block(node_ref[...] + attn_ref[...], n1g[...], n1b[...])
    f = _gelu_exact(
        jnp.dot(h, w1[...], preferred_element_type=jnp.float32, precision=jax.lax.Precision.HIGHEST) + b1[...]
    )
    f = jnp.dot(f, w2[...], preferred_element_type=jnp.float32, precision=jax.lax.Precision.HIGHEST) + b2[...]
    o_ref[...] = _ln_block(h + f, n2g[...], n2b[...])


def _layer_dense(node, attn, lp, bm=256):
    M = node.shape[0]
    H = _HID
    row = lambda v: v.reshape(1, -1)
    full = lambda shape: pl.BlockSpec(shape, lambda i: (0, 0))
    return pl.pallas_call(
        _layer_kern,
        grid=(pl.cdiv(M, bm),),
        in_specs=[
            pl.BlockSpec((bm, H), lambda i: (i, 0)),
            pl.BlockSpec((bm, H), lambda i: (i, 0)),
            full((1, H)),
            full((1, H)),
            full((H, 4 * H)),
            full((1, 4 * H)),
            full((4 * H, H)),
            full((1, H)),
            full((1, H)),
            full((1, H)),
        ],
        out_specs=pl.BlockSpec((bm, H), lambda i: (i, 0)),
        out_shape=jax.ShapeDtypeStruct((M, H), jnp.float32),
    )(
        node,
        attn,
        row(lp["n1g"]),
        row(lp["n1b"]),
        lp["fW1"],
        row(lp["fb1"]),
        lp["fW2"],
        row(lp["fb2"]),
        row(lp["n2g"]),
        row(lp["n2b"]),
    )


def _lnmm_kern(x_ref, g_ref, b_ref, w_ref, bw_ref, o_ref):
    h = _ln_block(x_ref[...], g_ref[...], b_ref[...])
    o_ref[...] = (
        jnp.dot(h, w_ref[...], preferred_element_type=jnp.float32, precision=jax.lax.Precision.HIGHEST) + bw_ref[...]
    )


def _ln_mm(x, g, b, W, bw, bm=512):
    M, K = x.shape
    N = W.shape[1]
    full = lambda shape: pl.BlockSpec(shape, lambda i: (0, 0))
    return pl.pallas_call(
        _lnmm_kern,
        grid=(pl.cdiv(M, bm),),
        in_specs=[
            pl.BlockSpec((bm, K), lambda i: (i, 0)),
            full((1, K)),
            full((1, K)),
            full((K, N)),
            full((1, N)),
        ],
        out_specs=pl.BlockSpec((bm, N), lambda i: (i, 0)),
        out_shape=jax.ShapeDtypeStruct((M, N), jnp.float32),
    )(x, g.reshape(1, K), b.reshape(1, K), W, bw.reshape(1, N))


def _escore_kern(xls_ref, xrd_ref, e_ref, att_ref, s_ref, o_ref):
    m = xls_ref[...] + xrd_ref[...] + e_ref[...]
    m = jnp.where(m >= 0.0, m, 0.2 * m)
    o_ref[...] = jnp.dot(
        m * att_ref[...],
        s_ref[...],
        preferred_element_type=jnp.float32,
        precision=jax.lax.Precision.HIGHEST,
    )


def _escore(xls, xrd, e2, att_flat, seg, bm=1024):
    M = xls.shape[0]
    H = _HID
    full = lambda shape: pl.BlockSpec(shape, lambda i: (0, 0))
    return pl.pallas_call(
        _escore_kern,
        grid=(pl.cdiv(M, bm),),
        in_specs=[
            pl.BlockSpec((bm, H), lambda i: (i, 0)),
            pl.BlockSpec((bm, H), lambda i: (i, 0)),
            pl.BlockSpec((bm, H), lambda i: (i, 0)),
            full((1, H)),
            full((H, _HEADS)),
        ],
        out_specs=pl.BlockSpec((bm, _HEADS), lambda i: (i, 0)),
        out_shape=jax.ShapeDtypeStruct((M, _HEADS), jnp.float32),
    )(xls, xrd, e2, att_flat, seg)


def _time_embed(t):
    half = _HID // 2
    tf = jnp.clip(t, 0, 999).astype(jnp.float32)
    freqs = jnp.exp(
        jnp.arange(half, dtype=jnp.float32) * -(math.log(10000.0) / (half - 1))
    )
    ang = tf[:, None] * freqs[None, :]
    return jnp.concatenate([jnp.sin(ang), jnp.cos(ang)], axis=-1)


def _gatv2(xn, src, dst, e_emb, p):
    xl2 = _mm_bias(xn, p["Wl"], p["bl"])
    xr2 = _mm_bias(xn, p["Wr"], p["br"])
    e2 = _mm_bias(e_emb, p["We"], jnp.zeros((_HID,), jnp.float32))
    xl = xl2.reshape(-1, _HEADS, _OUTC)
    xr = xr2.reshape(-1, _HEADS, _OUTC)
    e = e2.reshape(-1, _HEADS, _OUTC)
    seg = (
        (jnp.arange(_HID)[:, None] // _OUTC) == jnp.arange(_HEADS)[None, :]
    ).astype(jnp.float32)
    mask = src != dst
    mf = mask.astype(jnp.float32)
    cnt = jax.ops.segment_sum(mf, dst, num_segments=_N_NODES)
    loop_e = jax.ops.segment_sum(e * mf[:, None, None], dst, num_segments=_N_NODES)
    loop_e = jnp.where(
        cnt[:, None, None] > 0,
        loop_e / jnp.maximum(cnt, 1.0)[:, None, None],
        0.0,
    )
    m_loop = jax.nn.leaky_relu(xl + xr + loop_e, 0.2)
    a = _escore(xl2[src], xr2[dst], e2, p["att"].reshape(1, _HID), seg)
    a_loop = jnp.sum(m_loop * p["att"][None], axis=-1)
    a_for_max = jnp.where(mask[:, None], a, -jnp.inf)
    amax = jnp.maximum(
        jax.ops.segment_max(a_for_max, dst, num_segments=_N_NODES), a_loop
    )
    a_safe = jnp.where(mask[:, None], a, amax[dst])
    ex = jnp.exp(a_safe - amax[dst]) * mf[:, None]
    ex_loop = jnp.exp(a_loop - amax)
    den = jax.ops.segment_sum(ex, dst, num_segments=_N_NODES) + ex_loop
    out = jax.ops.segment_sum(
        xl[src] * (ex / (den[dst] + 1e-16))[..., None], dst, num_segments=_N_NODES
    )
    out = out + xl * (ex_loop / (den + 1e-16))[..., None]
    return out.reshape(-1, _HID) + p["bias"]


def kernel(x, edge_index, edge_attr, batch, timesteps, params):
    xi = jnp.clip(x, 0, _NODE_DIM - 1)
    ei = jnp.clip(edge_attr, 0, _EDGE_DIM - 1)
    node = params["node_embed"][xi]
    eemb = params["edge_embed"][ei]
    te = _time_embed(timesteps)
    te = (
        jax.nn.gelu(te @ params["tm_W1"] + params["tm_b1"], approximate=False)
        @ params["tm_W2"]
        + params["tm_b2"]
    )
    node = node + te[batch]
    src = edge_index[0]
    dst = edge_index[1]
    for lp in params["layers"]:
        attn = _gatv2(node, src, dst, eemb, lp)
        node = _layer_dense(node, attn, lp)
    pred_node = _ln_mm(node, params["hn_g"], params["hn_b"], params["hn_W"], params["hn_bW"])
    ef = (node[src] + node[dst]) * 0.5
    pred_edge = _ln_mm(ef, params["he_g"], params["he_b"], params["he_W"], params["he_bW"])
    return (pred_node, pred_edge)
